# baseline (speedup 1.0000x reference)
import functools

import jax
import jax.numpy as jnp
from jax import lax
from jax.experimental import pallas as pl
from jax.experimental.pallas import tpu as pltpu
from jax.experimental.pallas import tpu_sc as plsc


@functools.partial(jax.jit, static_argnums=(2, 3, 4))
def _sc_gather_t(table, idx, B, H, D):
    info = plsc.get_sparse_core_info()
    NC, NS = info.num_cores, info.num_subcores
    NW = NC * NS
    R = D // 8
    CB = B // 128
    G = CB // 4
    UNITS = H * G
    UPW = UNITS // NW
    C = 512
    HSTRIDE = R * CB * 1024
    RSTRIDE = CB * 1024

    mesh = plsc.VectorSubcoreMesh(core_axis_name="c", subcore_axis_name="s")

    @functools.partial(
        pl.kernel,
        mesh=mesh,
        out_type=jax.ShapeDtypeStruct((B * H * D,), jnp.float32),
        scratch_types=[
            pltpu.VMEM((UPW * C,), jnp.int32),
            pltpu.VMEM((C, D), jnp.float32),
            pltpu.VMEM((C, D), jnp.float32),
            pltpu.VMEM((C * D,), jnp.float32),
            pltpu.VMEM((C * D,), jnp.float32),
            pltpu.SemaphoreType.DMA,
            pltpu.SemaphoreType.DMA,
            pltpu.SemaphoreType.DMA,
            pltpu.SemaphoreType.DMA,
        ],
        compiler_params=pltpu.CompilerParams(
            use_tc_tiling_on_sc=False, needs_layout_passes=False),
    )
    def k(table_hbm, idx_hbm, out_hbm, idx_all, rows0, rows1, tr0, tr1,
          sg0, sg1, sw0, sw1):
        w = lax.axis_index("s") * NC + lax.axis_index("c")
        base_u = w * UPW
        pltpu.sync_copy(idx_hbm.at[pl.ds(w * (UPW * C), UPW * C)], idx_all)

        iota = lax.iota(jnp.int32, 16)

        rows = [rows0, rows1]
        trs = [tr0, tr1]
        sg = [sg0, sg1]
        sw = [sw0, sw1]

        def gather_desc(t, b):
            return pltpu.make_async_copy(
                table_hbm.at[idx_all.at[pl.ds(t * C, C)]], rows[b], sg[b])

        def write_descs(t, b):
            u = base_u + t
            h = u >> 5
            g = u & (G - 1)
            ds_ = []
            for r in range(R):
                flat0 = h * HSTRIDE + r * RSTRIDE + g * 4096
                ds_.append(pltpu.make_async_copy(
                    trs[b].at[pl.ds(r * 4096, 4096)],
                    out_hbm.at[pl.ds(flat0, 4096)], sw[b]))
            return ds_

        def transpose_unit(rows_b, tr_b):
            def abody(a, carry):
                colv = (iota + a) & 31
                storepat = ((colv >> 3) << 12) + ((colv & 7) << 7) + iota

                def body(blk, c2):
                    row_ids = iota + (blk << 4)
                    vals = plsc.load_gather(rows_b, [row_ids, colv])
                    sconst = ((blk >> 3) << 10) + ((blk & 7) << 4)
                    plsc.store_scatter(tr_b, [storepat + sconst], vals)
                    return c2

                lax.fori_loop(0, C // 16, body, 0, unroll=4)
                return carry

            lax.fori_loop(0, D, abody, 0)

        gather_desc(0, 0).start()

        def step(s, carry):
            for b in range(2):
                t = 2 * s + b
                nb = 1 - b
                @pl.when(t + 1 < UPW)
                def _():
                    gather_desc(t + 1, nb).start()
                gather_desc(t, b).wait()
                @pl.when(t >= 2)
                def _():
                    for d_ in write_descs(t - 2, b):
                        d_.wait()
                transpose_unit(rows[b], trs[b])
                for d_ in write_descs(t, b):
                    d_.start()
            return carry

        lax.fori_loop(0, UPW // 2, step, 0)
        for d_ in write_descs(UPW - 2, 0):
            d_.wait()
        for d_ in write_descs(UPW - 1, 1):
            d_.wait()

    return k(table, idx)


def _sc_depad(table):
    V, D = table.shape
    info = plsc.get_sparse_core_info()
    NW = info.num_cores * info.num_subcores
    RZ = 200
    NCHUNK = V // RZ
    NITER = (NCHUNK + NW - 1) // NW

    mesh = plsc.VectorSubcoreMesh(core_axis_name="c", subcore_axis_name="s")

    @functools.partial(
        pl.kernel,
        mesh=mesh,
        out_type=jax.ShapeDtypeStruct((V * D,), jnp.float32),
        scratch_types=[
            pltpu.VMEM((RZ, D), jnp.float32),
            pltpu.VMEM((RZ, D), jnp.float32),
            pltpu.VMEM((RZ * D,), jnp.float32),
            pltpu.SemaphoreType.DMA,
            pltpu.SemaphoreType.DMA,
            pltpu.SemaphoreType.DMA,
        ],
        compiler_params=pltpu.CompilerParams(
            use_tc_tiling_on_sc=True, needs_layout_passes=False),
    )
    def k(table_hbm, out_hbm, in0, in1, out_v, si0, si1, so):
        w = lax.axis_index("s") * info.num_cores + lax.axis_index("c")
        ins = [in0, in1]
        sis = [si0, si1]

        def in_desc(cid, b):
            return pltpu.make_async_copy(
                table_hbm.at[pl.ds(cid * RZ, RZ), :], ins[b], sis[b])

        @pl.when(w < NCHUNK)
        def _():
            in_desc(w, 0).start()

        def step(s, carry):
            for b in range(2):
                t = 2 * s + b
                cid = t * NW + w
                nid = cid + NW
                @pl.when(nid < NCHUNK)
                def _():
                    in_desc(nid, 1 - b).start()
                @pl.when(cid < NCHUNK)
                def _():
                    in_desc(cid, b).wait()

                    def body(q8, c2):
                        for kk in range(8):
                            q = q8 * 8 + kk
                            out_v[pl.ds(q * D, 16)] = ins[b][q, pl.ds(0, 16)]
                            out_v[pl.ds(q * D + 16, 16)] = \
                                ins[b][q, pl.ds(16, 16)]
                        return c2

                    lax.fori_loop(0, RZ // 8, body, 0, unroll=4)
                    pltpu.make_async_copy(
                        out_v, out_hbm.at[pl.ds(cid * (RZ * D), RZ * D)],
                        so).start()
                    pltpu.make_async_copy(
                        out_v, out_hbm.at[pl.ds(cid * (RZ * D), RZ * D)],
                        so).wait()
            return carry

        lax.fori_loop(0, (NITER + 1) // 2, step, 0)

    return k(table).reshape(V, D)


def kernel(X, table):
    B, H = X.shape
    V, D = table.shape
    idx = X.T.reshape(B * H).astype(jnp.int32)
    table_lin = _sc_depad(table)
    out_flat = _sc_gather_t(table_lin, idx, B, H, D)
    R = D // 8
    CB = B // 128
    out = (out_flat.reshape(H, R, CB, 8, 128)
           .transpose(2, 4, 0, 1, 3)
           .reshape(B, H, D))
    return out


# speedup vs baseline: 1.1093x; 1.1093x over previous
import functools

import jax
import jax.numpy as jnp
from jax import lax
from jax.experimental import pallas as pl
from jax.experimental.pallas import tpu as pltpu
from jax.experimental.pallas import tpu_sc as plsc


@functools.partial(jax.jit, static_argnums=(2, 3, 4))
def _sc_gather_t(table, idx, B, H, D):
    info = plsc.get_sparse_core_info()
    NC, NS = info.num_cores, info.num_subcores
    NW = NC * NS
    R = D // 8
    CB = B // 128
    G = CB // 4
    UNITS = H * G
    UPW = UNITS // NW
    C = 512
    HSTRIDE = R * CB * 1024
    RSTRIDE = CB * 1024

    mesh = plsc.VectorSubcoreMesh(core_axis_name="c", subcore_axis_name="s")

    @functools.partial(
        pl.kernel,
        mesh=mesh,
        out_type=jax.ShapeDtypeStruct((B * H * D,), jnp.float32),
        scratch_types=[
            pltpu.VMEM((UPW * C,), jnp.int32),
            pltpu.VMEM((C, D), jnp.float32),
            pltpu.VMEM((C, D), jnp.float32),
            pltpu.VMEM((C * D,), jnp.float32),
            pltpu.VMEM((C * D,), jnp.float32),
            pltpu.SemaphoreType.DMA,
            pltpu.SemaphoreType.DMA,
            pltpu.SemaphoreType.DMA,
            pltpu.SemaphoreType.DMA,
        ],
        compiler_params=pltpu.CompilerParams(
            use_tc_tiling_on_sc=False, needs_layout_passes=False),
    )
    def k(table_hbm, idx_hbm, out_hbm, idx_all, rows0, rows1, tr0, tr1,
          sg0, sg1, sw0, sw1):
        w = lax.axis_index("s") * NC + lax.axis_index("c")
        base_u = w * UPW
        pltpu.sync_copy(idx_hbm.at[pl.ds(w * (UPW * C), UPW * C)], idx_all)

        iota = lax.iota(jnp.int32, 16)

        rows = [rows0, rows1]
        trs = [tr0, tr1]
        sg = [sg0, sg1]
        sw = [sw0, sw1]

        def gather_desc(t, b):
            return pltpu.make_async_copy(
                table_hbm.at[idx_all.at[pl.ds(t * C, C)]], rows[b], sg[b])

        def write_descs(t, b):
            u = base_u + t
            h = u >> 5
            g = u & (G - 1)
            ds_ = []
            for r in range(R):
                flat0 = h * HSTRIDE + r * RSTRIDE + g * 4096
                ds_.append(pltpu.make_async_copy(
                    trs[b].at[pl.ds(r * 4096, 4096)],
                    out_hbm.at[pl.ds(flat0, 4096)], sw[b]))
            return ds_

        def transpose_unit(rows_b, tr_b):
            def abody(a, carry):
                colv = (iota + a) & 31
                storepat = ((colv >> 3) << 12) + ((colv & 7) << 7) + iota

                def body(blk, c2):
                    row_ids = iota + (blk << 4)
                    vals = plsc.load_gather(rows_b, [row_ids, colv])
                    sconst = ((blk >> 3) << 10) + ((blk & 7) << 4)
                    plsc.store_scatter(tr_b, [storepat + sconst], vals)
                    return c2

                lax.fori_loop(0, C // 16, body, 0, unroll=8)
                return carry

            lax.fori_loop(0, D, abody, 0)

        gather_desc(0, 0).start()

        def step(s, carry):
            for b in range(2):
                t = 2 * s + b
                nb = 1 - b
                @pl.when(t + 1 < UPW)
                def _():
                    gather_desc(t + 1, nb).start()
                gather_desc(t, b).wait()
                @pl.when(t >= 2)
                def _():
                    for d_ in write_descs(t - 2, b):
                        d_.wait()
                transpose_unit(rows[b], trs[b])
                for d_ in write_descs(t, b):
                    d_.start()
            return carry

        lax.fori_loop(0, UPW // 2, step, 0)
        for d_ in write_descs(UPW - 2, 0):
            d_.wait()
        for d_ in write_descs(UPW - 1, 1):
            d_.wait()

    return k(table, idx)


def kernel(X, table):
    B, H = X.shape
    V, D = table.shape
    idx = X.T.reshape(B * H).astype(jnp.int32)
    out_flat = _sc_gather_t(table, idx, B, H, D)
    R = D // 8
    CB = B // 128
    out = (out_flat.reshape(H, R, CB, 8, 128)
           .transpose(2, 4, 0, 1, 3)
           .reshape(B, H, D))
    return out


# speedup vs baseline: 1.1237x; 1.0130x over previous
import functools

import jax
import jax.numpy as jnp
from jax import lax
from jax.experimental import pallas as pl
from jax.experimental.pallas import tpu as pltpu
from jax.experimental.pallas import tpu_sc as plsc


@functools.partial(jax.jit, static_argnums=(2, 3, 4))
def _sc_gather_t(table, idx, B, H, D):
    info = plsc.get_sparse_core_info()
    NC, NS = info.num_cores, info.num_subcores
    NW = NC * NS
    R = D // 8
    CB = B // 128
    G = CB // 4
    UNITS = H * G
    UPW = UNITS // NW
    C = 512
    HSTRIDE = R * CB * 1024
    RSTRIDE = CB * 1024

    mesh = plsc.VectorSubcoreMesh(core_axis_name="c", subcore_axis_name="s")

    @functools.partial(
        pl.kernel,
        mesh=mesh,
        out_type=jax.ShapeDtypeStruct((B * H * D,), jnp.float32),
        scratch_types=[
            pltpu.VMEM((UPW * C,), jnp.int32),
            pltpu.VMEM((C, D), jnp.float32),
            pltpu.VMEM((C, D), jnp.float32),
            pltpu.VMEM((C * D,), jnp.float32),
            pltpu.VMEM((C * D,), jnp.float32),
            pltpu.SemaphoreType.DMA,
            pltpu.SemaphoreType.DMA,
            pltpu.SemaphoreType.DMA,
            pltpu.SemaphoreType.DMA,
        ],
        compiler_params=pltpu.CompilerParams(
            use_tc_tiling_on_sc=False, needs_layout_passes=False),
    )
    def k(table_hbm, idx_hbm, out_hbm, idx_all, rows0, rows1, tr0, tr1,
          sg0, sg1, sw0, sw1):
        w = lax.axis_index("s") * NC + lax.axis_index("c")
        base_u = w * UPW
        pltpu.sync_copy(idx_hbm.at[pl.ds(w * (UPW * C), UPW * C)], idx_all)

        iota = lax.iota(jnp.int32, 16)

        rows = [rows0, rows1]
        trs = [tr0, tr1]
        sg = [sg0, sg1]
        sw = [sw0, sw1]

        def gather_desc(t, b):
            return pltpu.make_async_copy(
                table_hbm.at[idx_all.at[pl.ds(t * C, C)]], rows[b], sg[b])

        def write_descs(t, b):
            u = base_u + t
            h = u >> 5
            g = u & (G - 1)
            ds_ = []
            for r in range(R):
                flat0 = h * HSTRIDE + r * RSTRIDE + g * 4096
                ds_.append(pltpu.make_async_copy(
                    trs[b].at[pl.ds(r * 4096, 4096)],
                    out_hbm.at[pl.ds(flat0, 4096)], sw[b]))
            return ds_

        def transpose_unit(rows_b, tr_b):
            def abody(a, carry):
                colv = (iota + a) & 31
                storepat = ((colv >> 3) << 12) + ((colv & 7) << 7) + iota

                def body(blk, c2):
                    row_ids = iota + (blk << 4)
                    vals = plsc.load_gather(rows_b, [row_ids, colv])
                    sconst = ((blk >> 3) << 10) + ((blk & 7) << 4)
                    plsc.store_scatter(tr_b, [storepat + sconst], vals)
                    return c2

                lax.fori_loop(0, C // 16, body, 0, unroll=16)
                return carry

            lax.fori_loop(0, D, abody, 0, unroll=2)

        gather_desc(0, 0).start()

        def step(s, carry):
            for b in range(2):
                t = 2 * s + b
                nb = 1 - b
                @pl.when(t + 1 < UPW)
                def _():
                    gather_desc(t + 1, nb).start()
                gather_desc(t, b).wait()
                @pl.when(t >= 2)
                def _():
                    for d_ in write_descs(t - 2, b):
                        d_.wait()
                transpose_unit(rows[b], trs[b])
                for d_ in write_descs(t, b):
                    d_.start()
            return carry

        lax.fori_loop(0, UPW // 2, step, 0)
        for d_ in write_descs(UPW - 2, 0):
            d_.wait()
        for d_ in write_descs(UPW - 1, 1):
            d_.wait()

    return k(table, idx)


def kernel(X, table):
    B, H = X.shape
    V, D = table.shape
    idx = X.T.reshape(B * H).astype(jnp.int32)
    out_flat = _sc_gather_t(table, idx, B, H, D)
    R = D // 8
    CB = B // 128
    out = (out_flat.reshape(H, R, CB, 8, 128)
           .transpose(2, 4, 0, 1, 3)
           .reshape(B, H, D))
    return out
